# Initial kernel scaffold; baseline (speedup 1.0000x reference)
import functools

import jax
import jax.numpy as jnp
from jax import lax
from jax.experimental import pallas as pl
from jax.experimental.pallas import tpu as pltpu
from jax.experimental.pallas import tpu_sc as plsc

N = 10000
E = 160000
D = 256
DH = 128
NSUB = 16
NCORE = 2

EPT = E // NSUB
CH = 128
NFULL = EPT // CH
TAIL = EPT - NFULL * CH

DEPT = E // (NSUB * NCORE)
DNFULL = DEPT // CH
DTAIL = DEPT - DNFULL * CH

_MESH = plsc.VectorSubcoreMesh(core_axis_name="c", subcore_axis_name="s")


def _deg_body(ei_hbm, z16_hbm, out_hbm, dbuf, dtbuf, ones_v, dacc):
    c = lax.axis_index("c")
    s = lax.axis_index("s")
    base = c * (NSUB * DEPT) + s * DEPT

    def _fill(r, _):
        ones_v[r] = jnp.ones((16,), jnp.float32)
        return _
    lax.fori_loop(0, CH, _fill, None)

    @pl.when(s == 0)
    def _init():
        pltpu.sync_copy(z16_hbm, dacc)

    plsc.subcore_barrier()

    def _chunk(j, _):
        pltpu.sync_copy(ei_hbm.at[1, pl.ds(base + j * CH, CH)], dbuf)
        pltpu.sync_copy(ones_v, dacc.at[dbuf], add=True)
        return _
    lax.fori_loop(0, DNFULL, _chunk, None)

    pltpu.sync_copy(ei_hbm.at[1, pl.ds(base + DNFULL * CH, DTAIL)], dtbuf)
    pltpu.sync_copy(ones_v.at[pl.ds(0, DTAIL)], dacc.at[dtbuf], add=True)

    plsc.subcore_barrier()

    @pl.when(s == 0)
    def _dump():
        pltpu.sync_copy(dacc, out_hbm.at[c])


_deg_call = pl.kernel(
    _deg_body,
    out_type=jax.ShapeDtypeStruct((NCORE, N, 16), jnp.float32),
    mesh=_MESH,
    scratch_types=[
        pltpu.VMEM((CH,), jnp.int32),
        pltpu.VMEM((DTAIL,), jnp.int32),
        pltpu.VMEM((CH, 16), jnp.float32),
        pltpu.VMEM_SHARED((N, 16), jnp.float32),
    ],
)


def _scat_body(y_hbm, ei_hbm, out_hbm, ibuf, rows, tbuf, trows, acc, gsem,
               tsem):
    c = lax.axis_index("c")
    s = lax.axis_index("s")
    coff = c * N
    ebase = s * EPT

    def load_idx(j, b):
        pltpu.sync_copy(ei_hbm.at[:, pl.ds(ebase + j * CH, CH)], ibuf.at[b])
        for v in range(CH // 16):
            sl = pl.ds(v * 16, 16)
            ibuf[b, 0, sl] = ibuf[b, 0, sl] + coff

    def start_gather(b):
        pltpu.async_copy(y_hbm.at[ibuf.at[b, 0]], rows.at[b], gsem.at[b])

    def wait_gather(b):
        pltpu.make_async_copy(y_hbm.at[ibuf.at[b, 0]], rows.at[b],
                              gsem.at[b]).wait()

    def scatter(b):
        pltpu.sync_copy(rows.at[b], acc.at[ibuf.at[b, 1]], add=True)

    load_idx(0, 0)
    start_gather(0)
    load_idx(1, 1)
    start_gather(1)

    @pl.when(s == 0)
    def _init():
        pltpu.sync_copy(y_hbm.at[pl.ds(coff, N)], acc)

    plsc.subcore_barrier()

    def _step(j2, _):
        for b in range(2):
            j = 2 * j2 + b
            wait_gather(b)
            scatter(b)
            load_idx(j + 2, b)
            start_gather(b)
        return _
    lax.fori_loop(0, NFULL // 2 - 1, _step, None)

    for b in range(2):
        wait_gather(b)
        scatter(b)

    pltpu.sync_copy(ei_hbm.at[:, pl.ds(ebase + NFULL * CH, TAIL)], tbuf)
    tbuf[0, :] = tbuf[0, :] + coff
    pltpu.async_copy(y_hbm.at[tbuf.at[0]], trows, tsem).wait()
    pltpu.sync_copy(trows, acc.at[tbuf.at[1]], add=True)

    plsc.subcore_barrier()

    @pl.when(s == 0)
    def _dump():
        pltpu.sync_copy(acc, out_hbm.at[c])


_scat_call = pl.kernel(
    _scat_body,
    out_type=jax.ShapeDtypeStruct((NCORE, N, DH), jnp.float32),
    mesh=_MESH,
    scratch_types=[
        pltpu.VMEM((2, 2, CH), jnp.int32),
        pltpu.VMEM((2, CH, DH), jnp.float32),
        pltpu.VMEM((2, TAIL), jnp.int32),
        pltpu.VMEM((TAIL, DH), jnp.float32),
        pltpu.VMEM_SHARED((N, DH), jnp.float32),
        pltpu.SemaphoreType.DMA((2,)),
        pltpu.SemaphoreType.DMA,
    ],
)


RB = 1000
GRID = N // RB


def _dinv_from_parts(dp_ref):
    deg = dp_ref[0, :, 0] + dp_ref[1, :, 0] + 1.0
    return lax.rsqrt(deg)


def _mm1_body(x_ref, w_ref, dp_ref, y_ref):
    dinv = _dinv_from_parts(dp_ref)
    xw = jnp.dot(x_ref[...], w_ref[...], preferred_element_type=jnp.float32)
    y = xw * dinv[:, None]
    y_ref[0] = y[:, :DH]
    y_ref[1] = y[:, DH:]


def _mm1(x, w1, degp):
    return pl.pallas_call(
        _mm1_body,
        grid=(GRID,),
        in_specs=[
            pl.BlockSpec((RB, D), lambda i: (i, 0)),
            pl.BlockSpec((D, D), lambda i: (0, 0)),
            pl.BlockSpec((NCORE, RB, 16), lambda i: (0, i, 0)),
        ],
        out_specs=pl.BlockSpec((NCORE, RB, DH), lambda i: (0, i, 0)),
        out_shape=jax.ShapeDtypeStruct((NCORE, N, DH), jnp.float32),
    )(x, w1, degp)


def _mm2_body(s_ref, b_ref, w_ref, dp_ref, y_ref):
    dinv = _dinv_from_parts(dp_ref)
    h = jnp.concatenate([s_ref[0], s_ref[1]], axis=1)
    h = h * dinv[:, None] + b_ref[...]
    xw = jnp.dot(h, w_ref[...], preferred_element_type=jnp.float32)
    y = xw * dinv[:, None]
    y_ref[0] = y[:, :DH]
    y_ref[1] = y[:, DH:]


def _mm2(s1, b1, w2, degp):
    return pl.pallas_call(
        _mm2_body,
        grid=(GRID,),
        in_specs=[
            pl.BlockSpec((NCORE, RB, DH), lambda i: (0, i, 0)),
            pl.BlockSpec((1, D), lambda i: (0, 0)),
            pl.BlockSpec((D, D), lambda i: (0, 0)),
            pl.BlockSpec((NCORE, RB, 16), lambda i: (0, i, 0)),
        ],
        out_specs=pl.BlockSpec((NCORE, RB, DH), lambda i: (0, i, 0)),
        out_shape=jax.ShapeDtypeStruct((NCORE, N, DH), jnp.float32),
    )(s1, b1.reshape(1, D), w2, degp)


def _fin_body(s_ref, b_ref, dp_ref, o_ref):
    dinv = _dinv_from_parts(dp_ref)
    h = jnp.concatenate([s_ref[0], s_ref[1]], axis=1)
    o_ref[...] = h * dinv[:, None] + b_ref[...]


def _fin(s2, b2, degp):
    return pl.pallas_call(
        _fin_body,
        grid=(GRID,),
        in_specs=[
            pl.BlockSpec((NCORE, RB, DH), lambda i: (0, i, 0)),
            pl.BlockSpec((1, D), lambda i: (0, 0)),
            pl.BlockSpec((NCORE, RB, 16), lambda i: (0, i, 0)),
        ],
        out_specs=pl.BlockSpec((RB, D), lambda i: (i, 0)),
        out_shape=jax.ShapeDtypeStruct((N, D), jnp.float32),
    )(s2, b2.reshape(1, D), degp)


def kernel(x, edge_index, W1, b1, W2, b2):
    z16 = jnp.zeros((N, 16), jnp.float32)
    degp = _deg_call(edge_index, z16)
    y1 = _mm1(x, W1, degp)
    s1 = _scat_call(y1.reshape(NCORE * N, DH), edge_index)
    y2 = _mm2(s1, b1, W2, degp)
    s2 = _scat_call(y2.reshape(NCORE * N, DH), edge_index)
    return _fin(s2, b2, degp)


# speedup vs baseline: 14.6375x; 14.6375x over previous
import functools

import jax
import jax.numpy as jnp
from jax import lax
from jax.experimental import pallas as pl
from jax.experimental.pallas import tpu as pltpu
from jax.experimental.pallas import tpu_sc as plsc

N = 10000
E = 160000
D = 256
DH = 128
NSUB = 16
NCORE = 2

EPT = E // NSUB
CH = 128
NFULL = EPT // CH
TAIL = EPT - NFULL * CH

DEPT = E // (NSUB * NCORE)
DNFULL = DEPT // CH
DTAIL = DEPT - DNFULL * CH

_MESH = plsc.VectorSubcoreMesh(core_axis_name="c", subcore_axis_name="s")


def _deg_body(dst_hbm, z16_hbm, ones_hbm, out_hbm, dbuf, dtbuf, ones_v, dacc):
    c = lax.axis_index("c")
    s = lax.axis_index("s")
    base = c * (NSUB * DEPT) + s * DEPT

    pltpu.sync_copy(ones_hbm, ones_v)

    @pl.when(s == 0)
    def _init():
        pltpu.sync_copy(z16_hbm, dacc)

    plsc.subcore_barrier()

    def _chunk(j, _):
        pltpu.sync_copy(dst_hbm.at[pl.ds(base + j * CH, CH)], dbuf)
        pltpu.sync_copy(ones_v, dacc.at[dbuf], add=True)
        return _
    lax.fori_loop(0, DNFULL, _chunk, None)

    pltpu.sync_copy(dst_hbm.at[pl.ds(base + DNFULL * CH, DTAIL)], dtbuf)
    pltpu.sync_copy(ones_v.at[pl.ds(0, DTAIL)], dacc.at[dtbuf], add=True)

    plsc.subcore_barrier()

    @pl.when(s == 0)
    def _dump():
        pltpu.sync_copy(dacc, out_hbm.at[c])


_deg_call = pl.kernel(
    _deg_body,
    out_type=jax.ShapeDtypeStruct((NCORE, N, 16), jnp.float32),
    mesh=_MESH,
    scratch_types=[
        pltpu.VMEM((CH,), jnp.int32),
        pltpu.VMEM((DTAIL,), jnp.int32),
        pltpu.VMEM((CH, 16), jnp.float32),
        pltpu.VMEM_SHARED((N, 16), jnp.float32),
    ],
)


def _scat_body(y_hbm, src_hbm, dst_hbm, out_hbm, ibuf, rows, tbuf, trows, acc,
               gsem, tsem):
    c = lax.axis_index("c")
    s = lax.axis_index("s")
    coff = c * N
    ebase = s * EPT

    def load_idx(j, b):
        pltpu.sync_copy(src_hbm.at[pl.ds(ebase + j * CH, CH)], ibuf.at[b, 0])
        pltpu.sync_copy(dst_hbm.at[pl.ds(ebase + j * CH, CH)], ibuf.at[b, 1])
        for v in range(CH // 16):
            sl = pl.ds(v * 16, 16)
            ibuf[b, 0, sl] = ibuf[b, 0, sl] + coff

    def start_gather(b):
        pltpu.async_copy(y_hbm.at[ibuf.at[b, 0]], rows.at[b], gsem.at[b])

    def wait_gather(b):
        pltpu.make_async_copy(y_hbm.at[ibuf.at[b, 0]], rows.at[b],
                              gsem.at[b]).wait()

    def scatter(b):
        pltpu.sync_copy(rows.at[b], acc.at[ibuf.at[b, 1]], add=True)

    load_idx(0, 0)
    start_gather(0)
    load_idx(1, 1)
    start_gather(1)

    @pl.when(s == 0)
    def _init():
        pltpu.sync_copy(y_hbm.at[pl.ds(coff, N)], acc)

    plsc.subcore_barrier()

    def _step(j2, _):
        for b in range(2):
            j = 2 * j2 + b
            wait_gather(b)
            scatter(b)
            load_idx(j + 2, b)
            start_gather(b)
        return _
    lax.fori_loop(0, NFULL // 2 - 1, _step, None)

    for b in range(2):
        wait_gather(b)
        scatter(b)

    pltpu.sync_copy(src_hbm.at[pl.ds(ebase + NFULL * CH, TAIL)], tbuf.at[0])
    pltpu.sync_copy(dst_hbm.at[pl.ds(ebase + NFULL * CH, TAIL)], tbuf.at[1])
    tbuf[0, :] = tbuf[0, :] + coff
    pltpu.async_copy(y_hbm.at[tbuf.at[0]], trows, tsem).wait()
    pltpu.sync_copy(trows, acc.at[tbuf.at[1]], add=True)

    plsc.subcore_barrier()

    @pl.when(s == 0)
    def _dump():
        pltpu.sync_copy(acc, out_hbm.at[c])


_scat_call = pl.kernel(
    _scat_body,
    out_type=jax.ShapeDtypeStruct((NCORE, N, DH), jnp.float32),
    mesh=_MESH,
    scratch_types=[
        pltpu.VMEM((2, 2, CH), jnp.int32),
        pltpu.VMEM((2, CH, DH), jnp.float32),
        pltpu.VMEM((2, TAIL), jnp.int32),
        pltpu.VMEM((TAIL, DH), jnp.float32),
        pltpu.VMEM_SHARED((N, DH), jnp.float32),
        pltpu.SemaphoreType.DMA((2,)),
        pltpu.SemaphoreType.DMA,
    ],
)


RB = 1000
GRID = N // RB


def _dinv_from_parts(dp_ref):
    deg = dp_ref[0, :, 0] + dp_ref[1, :, 0] + 1.0
    return lax.rsqrt(deg)


def _mm1_body(x_ref, w_ref, dp_ref, y_ref):
    dinv = _dinv_from_parts(dp_ref)
    xw = jnp.dot(x_ref[...], w_ref[...], preferred_element_type=jnp.float32)
    y = xw * dinv[:, None]
    y_ref[0] = y[:, :DH]
    y_ref[1] = y[:, DH:]


def _mm1(x, w1, degp):
    return pl.pallas_call(
        _mm1_body,
        grid=(GRID,),
        in_specs=[
            pl.BlockSpec((RB, D), lambda i: (i, 0)),
            pl.BlockSpec((D, D), lambda i: (0, 0)),
            pl.BlockSpec((NCORE, RB, 16), lambda i: (0, i, 0)),
        ],
        out_specs=pl.BlockSpec((NCORE, RB, DH), lambda i: (0, i, 0)),
        out_shape=jax.ShapeDtypeStruct((NCORE, N, DH), jnp.float32),
    )(x, w1, degp)


def _mm2_body(s_ref, b_ref, w_ref, dp_ref, y_ref):
    dinv = _dinv_from_parts(dp_ref)
    h = jnp.concatenate([s_ref[0], s_ref[1]], axis=1)
    h = h * dinv[:, None] + b_ref[...]
    xw = jnp.dot(h, w_ref[...], preferred_element_type=jnp.float32)
    y = xw * dinv[:, None]
    y_ref[0] = y[:, :DH]
    y_ref[1] = y[:, DH:]


def _mm2(s1, b1, w2, degp):
    return pl.pallas_call(
        _mm2_body,
        grid=(GRID,),
        in_specs=[
            pl.BlockSpec((NCORE, RB, DH), lambda i: (0, i, 0)),
            pl.BlockSpec((1, D), lambda i: (0, 0)),
            pl.BlockSpec((D, D), lambda i: (0, 0)),
            pl.BlockSpec((NCORE, RB, 16), lambda i: (0, i, 0)),
        ],
        out_specs=pl.BlockSpec((NCORE, RB, DH), lambda i: (0, i, 0)),
        out_shape=jax.ShapeDtypeStruct((NCORE, N, DH), jnp.float32),
    )(s1, b1.reshape(1, D), w2, degp)


def _fin_body(s_ref, b_ref, dp_ref, o_ref):
    dinv = _dinv_from_parts(dp_ref)
    h = jnp.concatenate([s_ref[0], s_ref[1]], axis=1)
    o_ref[...] = h * dinv[:, None] + b_ref[...]


def _fin(s2, b2, degp):
    return pl.pallas_call(
        _fin_body,
        grid=(GRID,),
        in_specs=[
            pl.BlockSpec((NCORE, RB, DH), lambda i: (0, i, 0)),
            pl.BlockSpec((1, D), lambda i: (0, 0)),
            pl.BlockSpec((NCORE, RB, 16), lambda i: (0, i, 0)),
        ],
        out_specs=pl.BlockSpec((RB, D), lambda i: (i, 0)),
        out_shape=jax.ShapeDtypeStruct((N, D), jnp.float32),
    )(s2, b2.reshape(1, D), degp)


def kernel(x, edge_index, W1, b1, W2, b2):
    src = edge_index[0]
    dst = edge_index[1]
    z16 = jnp.zeros((N, 16), jnp.float32)
    ones = jnp.ones((CH, 16), jnp.float32)
    degp = _deg_call(dst, z16, ones)
    y1 = _mm1(x, W1, degp)
    s1 = _scat_call(y1.reshape(NCORE * N, DH), src, dst)
    y2 = _mm2(s1, b1, W2, degp)
    s2 = _scat_call(y2.reshape(NCORE * N, DH), src, dst)
    return _fin(s2, b2, degp)


# speedup vs baseline: 20.4853x; 1.3995x over previous
import jax
import jax.numpy as jnp
from jax import lax
from jax.experimental import pallas as pl
from jax.experimental.pallas import tpu as pltpu
from jax.experimental.pallas import tpu_sc as plsc

N = 10000
E = 160000
D = 256
DH = 128
NSUB = 16
NCORE = 2

EPT = E // NSUB
CH = 80
NCH = EPT // CH
NBUF = 4
NBUFI = 5

DCH = 40
DEPT = E // (NSUB * NCORE)
DNCH = DEPT // DCH

_MESH = plsc.VectorSubcoreMesh(core_axis_name="c", subcore_axis_name="s")


def _deg_body(dst3d_hbm, z16_hbm, ones_hbm, out_hbm, dstv, ones_v, dacc, ssem):
    c = lax.axis_index("c")
    s = lax.axis_index("s")
    w = c * NSUB + s

    pltpu.sync_copy(ones_hbm, ones_v)
    pltpu.sync_copy(dst3d_hbm.at[w], dstv)

    @pl.when(s < 15)
    def _inita():
        pltpu.sync_copy(z16_hbm.at[pl.ds(s * 624, 624)],
                        dacc.at[pl.ds(s * 624, 624)])

    @pl.when(s == 15)
    def _initb():
        pltpu.sync_copy(z16_hbm.at[pl.ds(9360, 640)],
                        dacc.at[pl.ds(9360, 640)])

    plsc.subcore_barrier()

    def _chunk(j, _):
        pltpu.async_copy(ones_v, dacc.at[dstv.at[j]], ssem, add=True)
        return _
    lax.fori_loop(0, DNCH, _chunk, None)

    def _drain(j, _):
        pltpu.make_async_copy(ones_v, dacc.at[dstv.at[0]], ssem).wait()
        return _
    lax.fori_loop(0, DNCH, _drain, None)

    plsc.subcore_barrier()

    @pl.when(s < 15)
    def _dumpa():
        pltpu.sync_copy(dacc.at[pl.ds(s * 624, 624)],
                        out_hbm.at[c].at[pl.ds(s * 624, 624)])

    @pl.when(s == 15)
    def _dumpb():
        pltpu.sync_copy(dacc.at[pl.ds(9360, 640)],
                        out_hbm.at[c].at[pl.ds(9360, 640)])


_deg_call = pl.kernel(
    _deg_body,
    out_type=jax.ShapeDtypeStruct((NCORE, N, 16), jnp.float32),
    mesh=_MESH,
    scratch_types=[
        pltpu.VMEM((DNCH, DCH), jnp.int32),
        pltpu.VMEM((DCH, 16), jnp.float32),
        pltpu.VMEM_SHARED((N, 16), jnp.float32),
        pltpu.SemaphoreType.DMA,
    ],
)


def _scat_body(y_hbm, src_hbm, dst_hbm, out_hbm, idx, rows, acc, isem, gsem,
               ssem):
    c = lax.axis_index("c")
    s = lax.axis_index("s")
    coff = c * N
    ebase = s * EPT

    def idx_load(j):
        b = j % NBUFI
        sl = pl.ds(ebase + j * CH, CH)
        pltpu.async_copy(src_hbm.at[sl], idx.at[b, 0], isem.at[b])
        pltpu.async_copy(dst_hbm.at[sl], idx.at[b, 1], isem.at[b])

    def wait_idx_fix(j):
        b = j % NBUFI
        sl = pl.ds(ebase + j * CH, CH)
        pltpu.make_async_copy(src_hbm.at[sl], idx.at[b, 0], isem.at[b]).wait()
        pltpu.make_async_copy(dst_hbm.at[sl], idx.at[b, 1], isem.at[b]).wait()
        for k in range(CH // 16):
            ksl = pl.ds(k * 16, 16)
            idx[b, 0, ksl] = idx[b, 0, ksl] + coff

    def start_gather(j):
        pltpu.async_copy(y_hbm.at[idx.at[j % NBUFI, 0]], rows.at[j % NBUF],
                         gsem.at[j % NBUF])

    def wait_gather(j):
        pltpu.make_async_copy(y_hbm.at[idx.at[j % NBUFI, 0]],
                              rows.at[j % NBUF], gsem.at[j % NBUF]).wait()

    def start_scatter(j):
        pltpu.async_copy(rows.at[j % NBUF], acc.at[idx.at[j % NBUFI, 1]],
                         ssem.at[j % NBUF], add=True)

    def wait_scatter_buf(b):
        pltpu.make_async_copy(rows.at[b], acc.at[idx.at[0, 1]],
                              ssem.at[b]).wait()

    idx_load(0)
    idx_load(1)
    idx_load(2)

    @pl.when(s < 15)
    def _inita():
        pltpu.sync_copy(y_hbm.at[pl.ds(coff + s * 624, 624)],
                        acc.at[pl.ds(s * 624, 624)])

    @pl.when(s == 15)
    def _initb():
        pltpu.sync_copy(y_hbm.at[pl.ds(coff + 9360, 640)],
                        acc.at[pl.ds(9360, 640)])

    plsc.subcore_barrier()

    wait_idx_fix(0)
    start_gather(0)
    wait_idx_fix(1)
    start_gather(1)

    def step(j, refill, swait):
        if swait:
            wait_scatter_buf((j + 2) % NBUF)
        if refill:
            idx_load(j + 3)
        wait_idx_fix(j + 2)
        start_gather(j + 2)
        wait_gather(j)
        start_scatter(j)

    step(0, refill=True, swait=False)
    step(1, refill=True, swait=False)

    def _loop(o, _):
        for k in range(NBUF):
            step(2 + o * NBUF + k, refill=True, swait=True)
        return _
    lax.fori_loop(0, (NCH - 5) // NBUF, _loop, None)

    step(NCH - 3, refill=False, swait=True)
    wait_gather(NCH - 2)
    start_scatter(NCH - 2)
    wait_gather(NCH - 1)
    start_scatter(NCH - 1)

    for b in range(NBUF):
        wait_scatter_buf(b)

    plsc.subcore_barrier()

    @pl.when(s < 15)
    def _dumpa():
        pltpu.sync_copy(acc.at[pl.ds(s * 624, 624)],
                        out_hbm.at[c].at[pl.ds(s * 624, 624)])

    @pl.when(s == 15)
    def _dumpb():
        pltpu.sync_copy(acc.at[pl.ds(9360, 640)],
                        out_hbm.at[c].at[pl.ds(9360, 640)])


_scat_call = pl.kernel(
    _scat_body,
    out_type=jax.ShapeDtypeStruct((NCORE, N, DH), jnp.float32),
    mesh=_MESH,
    scratch_types=[
        pltpu.VMEM((NBUFI, 2, CH), jnp.int32),
        pltpu.VMEM((NBUF, CH, DH), jnp.float32),
        pltpu.VMEM_SHARED((N, DH), jnp.float32),
        pltpu.SemaphoreType.DMA((NBUFI,)),
        pltpu.SemaphoreType.DMA((NBUF,)),
        pltpu.SemaphoreType.DMA((NBUF,)),
    ],
)


RB = 1000
GRID = N // RB


def _dinv_from_parts(dp_ref):
    i = pl.program_id(0)
    sl = pl.ds(i * RB, RB)
    deg = dp_ref[0, sl, 0] + dp_ref[1, sl, 0] + 1.0
    return lax.rsqrt(deg)


def _mm1_body(x_ref, w_ref, dp_ref, y_ref):
    dinv = _dinv_from_parts(dp_ref)
    xw = jnp.dot(x_ref[...], w_ref[...], preferred_element_type=jnp.float32)
    y = xw * dinv[:, None]
    y_ref[0] = y[:, :DH]
    y_ref[1] = y[:, DH:]


def _mm1(x, w1, degp):
    return pl.pallas_call(
        _mm1_body,
        grid=(GRID,),
        in_specs=[
            pl.BlockSpec((RB, D), lambda i: (i, 0)),
            pl.BlockSpec((D, D), lambda i: (0, 0)),
            pl.BlockSpec((NCORE, N, 16), lambda i: (0, 0, 0)),
        ],
        out_specs=pl.BlockSpec((NCORE, RB, DH), lambda i: (0, i, 0)),
        out_shape=jax.ShapeDtypeStruct((NCORE, N, DH), jnp.float32),
    )(x, w1, degp)


def _mm2_body(s_ref, b_ref, w_ref, dp_ref, y_ref):
    dinv = _dinv_from_parts(dp_ref)
    h = jnp.concatenate([s_ref[0], s_ref[1]], axis=1)
    h = h * dinv[:, None] + b_ref[...]
    xw = jnp.dot(h, w_ref[...], preferred_element_type=jnp.float32)
    y = xw * dinv[:, None]
    y_ref[0] = y[:, :DH]
    y_ref[1] = y[:, DH:]


def _mm2(s1, b1, w2, degp):
    return pl.pallas_call(
        _mm2_body,
        grid=(GRID,),
        in_specs=[
            pl.BlockSpec((NCORE, RB, DH), lambda i: (0, i, 0)),
            pl.BlockSpec((1, D), lambda i: (0, 0)),
            pl.BlockSpec((D, D), lambda i: (0, 0)),
            pl.BlockSpec((NCORE, N, 16), lambda i: (0, 0, 0)),
        ],
        out_specs=pl.BlockSpec((NCORE, RB, DH), lambda i: (0, i, 0)),
        out_shape=jax.ShapeDtypeStruct((NCORE, N, DH), jnp.float32),
    )(s1, b1.reshape(1, D), w2, degp)


def _fin_body(s_ref, b_ref, dp_ref, o_ref):
    dinv = _dinv_from_parts(dp_ref)
    h = jnp.concatenate([s_ref[0], s_ref[1]], axis=1)
    o_ref[...] = h * dinv[:, None] + b_ref[...]


def _fin(s2, b2, degp):
    return pl.pallas_call(
        _fin_body,
        grid=(GRID,),
        in_specs=[
            pl.BlockSpec((NCORE, RB, DH), lambda i: (0, i, 0)),
            pl.BlockSpec((1, D), lambda i: (0, 0)),
            pl.BlockSpec((NCORE, N, 16), lambda i: (0, 0, 0)),
        ],
        out_specs=pl.BlockSpec((RB, D), lambda i: (i, 0)),
        out_shape=jax.ShapeDtypeStruct((N, D), jnp.float32),
    )(s2, b2.reshape(1, D), degp)


def kernel(x, edge_index, W1, b1, W2, b2):
    src = edge_index[0]
    dst = edge_index[1]
    dst3d_deg = dst.reshape(NSUB * NCORE, DNCH, DCH)
    z16 = jnp.zeros((N, 16), jnp.float32)
    ones = jnp.ones((DCH, 16), jnp.float32)
    degp = _deg_call(dst3d_deg, z16, ones)
    y1 = _mm1(x, W1, degp)
    s1 = _scat_call(y1.reshape(NCORE * N, DH), src, dst)
    y2 = _mm2(s1, b1, W2, degp)
    s2 = _scat_call(y2.reshape(NCORE * N, DH), src, dst)
    return _fin(s2, b2, degp)
